# baseline (speedup 1.0000x reference)
import functools

import jax
import jax.numpy as jnp
from jax.experimental import pallas as pl
from jax.experimental.pallas import tpu as pltpu


def _lcam_kernel(x_ref, w1_ref, w2_ref, o_ref, *, inv_hw):
    xb = x_ref[0]
    mh = jnp.max(xb, axis=1)
    sh = jnp.sum(xb, axis=1)
    pmax = jnp.max(mh, axis=-1, keepdims=True)
    pavg = jnp.sum(sh, axis=-1, keepdims=True) * inv_hw
    p2 = jnp.concatenate([pmax, pavg], axis=1)
    h = jnp.maximum(
        jnp.dot(w1_ref[...], p2, preferred_element_type=jnp.float32), 0.0)
    ht = (h[:, 0:1] + h[:, 1:2]).T
    y = jax.lax.dot_general(
        ht, w2_ref[...],
        dimension_numbers=(((1,), (1,)), ((), ())),
        preferred_element_type=jnp.float32)
    o_ref[...] = jax.nn.sigmoid(y)[None]


@jax.jit
def _lcam(x, w1, w2):
    B, C, H, W = x.shape
    C_ = w1.shape[0]

    x3 = x
    w1m = w1.reshape(C_, C)
    w2m = w2.reshape(C, C_)

    out = pl.pallas_call(
        functools.partial(_lcam_kernel, inv_hw=1.0 / (H * W)),
        out_shape=jax.ShapeDtypeStruct((B, 1, C), jnp.float32),
        grid=(B,),
        in_specs=[
            pl.BlockSpec((1, C, H, W), lambda i: (i, 0, 0, 0)),
            pl.BlockSpec((C_, C), lambda i: (0, 0)),
            pl.BlockSpec((C, C_), lambda i: (0, 0)),
        ],
        out_specs=pl.BlockSpec((1, 1, C), lambda i: (i, 0, 0)),
        compiler_params=pltpu.CompilerParams(
            dimension_semantics=("parallel",),
            vmem_limit_bytes=64 * 1024 * 1024),
    )(x3, w1m, w2m)

    return out.reshape(B, C, 1, 1).astype(x.dtype)


def kernel(x, w1, w2):
    return _lcam(x, w1, w2)


# speedup vs baseline: 7.8528x; 7.8528x over previous
import functools

import jax
import jax.numpy as jnp
from jax.experimental import pallas as pl
from jax.experimental.pallas import tpu as pltpu


def _lcam_kernel(x_ref, w1_ref, w2_ref, o_ref, *, inv_hw):
    xb = x_ref[0]
    pmax = jnp.max(xb, axis=0, keepdims=True)
    pavg = jnp.sum(xb, axis=0, keepdims=True) * inv_hw
    p2 = jnp.concatenate([pmax, pavg], axis=0)
    h = jax.lax.dot_general(
        p2, w1_ref[...],
        dimension_numbers=(((1,), (1,)), ((), ())),
        preferred_element_type=jnp.float32)
    h = jnp.maximum(h, 0.0)
    hrow = h[0:1] + h[1:2]
    y = jax.lax.dot_general(
        hrow, w2_ref[...],
        dimension_numbers=(((1,), (1,)), ((), ())),
        preferred_element_type=jnp.float32)
    o_ref[...] = jax.nn.sigmoid(y)[None]


@jax.jit
def _lcam(x, w1, w2):
    B, C, H, W = x.shape
    C_ = w1.shape[0]
    HW = H * W

    xt = jnp.transpose(x, (0, 2, 3, 1)).reshape(B, HW, C)
    w1m = w1.reshape(C_, C)
    w2m = w2.reshape(C, C_)

    out = pl.pallas_call(
        functools.partial(_lcam_kernel, inv_hw=1.0 / HW),
        out_shape=jax.ShapeDtypeStruct((B, 1, C), jnp.float32),
        grid=(B,),
        in_specs=[
            pl.BlockSpec((1, HW, C), lambda i: (i, 0, 0)),
            pl.BlockSpec((C_, C), lambda i: (0, 0)),
            pl.BlockSpec((C, C_), lambda i: (0, 0)),
        ],
        out_specs=pl.BlockSpec((1, 1, C), lambda i: (i, 0, 0)),
        compiler_params=pltpu.CompilerParams(
            dimension_semantics=("parallel",),
            vmem_limit_bytes=64 * 1024 * 1024),
    )(xt, w1m, w2m)

    return out.reshape(B, C, 1, 1).astype(x.dtype)


def kernel(x, w1, w2):
    return _lcam(x, w1, w2)


# speedup vs baseline: 12.1026x; 1.5412x over previous
import functools

import jax
import jax.numpy as jnp
from jax.experimental import pallas as pl
from jax.experimental.pallas import tpu as pltpu


def _lcam_kernel(x_ref, w1_ref, w2_ref, o_ref, *, inv_hw, nb):
    xb = x_ref[...]
    pmax = jnp.max(xb, axis=1)
    pavg = jnp.sum(xb, axis=1) * inv_hw
    p2 = jnp.concatenate([pmax, pavg], axis=0)
    h = jax.lax.dot_general(
        p2, w1_ref[...],
        dimension_numbers=(((1,), (1,)), ((), ())),
        preferred_element_type=jnp.float32)
    h = jnp.maximum(h, 0.0)
    hrow = h[:nb] + h[nb:]
    y = jax.lax.dot_general(
        hrow, w2_ref[...],
        dimension_numbers=(((1,), (1,)), ((), ())),
        preferred_element_type=jnp.float32)
    o_ref[...] = jax.nn.sigmoid(y)[:, None, :]


@jax.jit
def _lcam(x, w1, w2):
    B, C, H, W = x.shape
    C_ = w1.shape[0]
    HW = H * W

    xt = jnp.transpose(x, (0, 2, 3, 1)).reshape(B, HW, C)
    w1m = w1.reshape(C_, C)
    w2m = w2.reshape(C, C_)

    nb = 4
    out = pl.pallas_call(
        functools.partial(_lcam_kernel, inv_hw=1.0 / HW, nb=nb),
        out_shape=jax.ShapeDtypeStruct((B, 1, C), jnp.float32),
        grid=(B // nb,),
        in_specs=[
            pl.BlockSpec((nb, HW, C), lambda i: (i, 0, 0)),
            pl.BlockSpec((C_, C), lambda i: (0, 0)),
            pl.BlockSpec((C, C_), lambda i: (0, 0)),
        ],
        out_specs=pl.BlockSpec((nb, 1, C), lambda i: (i, 0, 0)),
        compiler_params=pltpu.CompilerParams(
            dimension_semantics=("parallel",),
            vmem_limit_bytes=64 * 1024 * 1024),
    )(xt, w1m, w2m)

    return out.reshape(B, C, 1, 1).astype(x.dtype)


def kernel(x, w1, w2):
    return _lcam(x, w1, w2)
